# Initial kernel scaffold; baseline (speedup 1.0000x reference)
import jax
import jax.numpy as jnp
from jax.experimental import pallas as pl


def kernel(maps, edge_index):
    raise NotImplementedError("write your pallas kernel here")


# speedup vs baseline: 2.3184x; 2.3184x over previous
import functools

import jax
import jax.numpy as jnp
from jax import lax
from jax.experimental import pallas as pl
from jax.experimental.pallas import tpu as pltpu
from jax.experimental.pallas import tpu_sc as plsc

NUM_NODES = 50000
D = 4
NC, NS, LANES = 2, 16, 16
NW = NC * NS

RPT = 3136
SIZE_P = NS * RPT

KC = 39
CE = KC * 128
KCD = 13
CED = KCD * 128


def _lane_rc():
    lane = lax.iota(jnp.int32, 16)
    return lax.shift_right_logical(lane, 2), jnp.bitwise_and(lane, 3)


def _sq_loop(ref, n16):
    roff, coff = _lane_rc()

    def body(i, c):
        rows = i * 4 + roff
        x = plsc.load_gather(ref, [rows, coff])
        plsc.store_scatter(ref, [rows, coff], x * x)
        return c
    lax.fori_loop(0, n16, body, 0)


def _scatter_body(e3, maps_in, part, idx_v, vals_v, acc_sh):
    nrows = e3.shape[1]
    rb = nrows // NW
    rem = nrows - rb * NW
    cid = lax.axis_index("c")
    sid = lax.axis_index("s")
    wid = sid * NC + cid
    roff, coff = _lane_rc()

    def zb(i, c):
        plsc.store_scatter(vals_v, [i * 4 + roff, coff],
                           jnp.zeros((16,), jnp.float32))
        return c
    lax.fori_loop(0, (RPT * D) // 16, zb, 0)
    pltpu.sync_copy(vals_v.at[pl.ds(0, RPT)], acc_sh.at[pl.ds(sid * RPT, RPT)])
    plsc.subcore_barrier()

    r0 = wid * rb + jnp.minimum(wid, rem)

    def chunk(c, carry):
        r = r0 + c * KC
        pltpu.sync_copy(e3.at[0, pl.ds(r, KC)], idx_v)
        pltpu.sync_copy(maps_in.at[pl.ds(r * 128, CE)], vals_v)
        _sq_loop(vals_v, (CE * D) // 16)
        for j in range(KC):
            pltpu.sync_copy(vals_v.at[pl.ds(j * 128, 128)],
                            acc_sh.at[idx_v.at[j]], add=True)
        return carry
    lax.fori_loop(0, rb // KC, chunk, 0)

    @pl.when(wid < rem)
    def _extra():
        r = r0 + rb
        pltpu.sync_copy(e3.at[0, pl.ds(r, 1)], idx_v.at[pl.ds(0, 1)])
        pltpu.sync_copy(maps_in.at[pl.ds(r * 128, 128)],
                        vals_v.at[pl.ds(0, 128)])
        _sq_loop(vals_v, (128 * D) // 16)
        pltpu.sync_copy(vals_v.at[pl.ds(0, 128)],
                        acc_sh.at[idx_v.at[0]], add=True)

    plsc.subcore_barrier()
    pltpu.sync_copy(acc_sh.at[pl.ds(sid * RPT, RPT)], vals_v.at[pl.ds(0, RPT)])
    pltpu.sync_copy(vals_v.at[pl.ds(0, RPT)], part.at[cid, pl.ds(sid * RPT, RPT)])


def _norm_body(p_ref, dsi_ref, dn_ref):
    d = p_ref[0] + p_ref[1]
    dp1 = d + 1.0
    dsi_ref[...] = lax.rsqrt(dp1)
    dn_ref[...] = d / dp1


def _gather_body(e3, maps_in, dsi, tril, idxr_v, idxc_v, bufr, bufc, bufa, bufb,
                 semr, semc):
    half_e = maps_in.shape[0] // 2
    nrows = half_e // 128
    rb = nrows // NW
    rem = nrows - rb * NW
    cid = lax.axis_index("c")
    sid = lax.axis_index("s")
    wid = sid * NC + cid
    roff, coff = _lane_rc()

    r0 = wid * rb + jnp.minimum(wid, rem)

    def mul_loop(n16):
        def body(i, c):
            rows = i * 4 + roff
            a = plsc.load_gather(bufa, [rows, coff])
            b = plsc.load_gather(bufb, [rows, coff])
            rr = plsc.load_gather(bufr, [rows, coff])
            cc = plsc.load_gather(bufc, [rows, coff])
            plsc.store_scatter(bufa, [rows, coff], -(a * b) * rr * cc)
            return c
        lax.fori_loop(0, n16, body, 0)

    def chunk(c, carry):
        r = r0 + c * KCD
        pltpu.sync_copy(e3.at[0, pl.ds(r, KCD)], idxr_v)
        pltpu.sync_copy(e3.at[1, pl.ds(r, KCD)], idxc_v)
        cps = []
        for j in range(KCD):
            cps.append(pltpu.async_copy(
                dsi.at[idxr_v.at[j]], bufr.at[pl.ds(j * 128, 128)], semr))
            cps.append(pltpu.async_copy(
                dsi.at[idxc_v.at[j]], bufc.at[pl.ds(j * 128, 128)], semc))
        pltpu.sync_copy(maps_in.at[pl.ds(r * 128, CED)], bufa)
        pltpu.sync_copy(maps_in.at[pl.ds(half_e + r * 128, CED)], bufb)
        for cp in cps:
            cp.wait()
        mul_loop((CED * D) // 16)
        pltpu.sync_copy(bufa, tril.at[pl.ds(r * 128, CED)])
        return carry
    lax.fori_loop(0, rb // KCD, chunk, 0)

    @pl.when(wid < rem)
    def _extra():
        r = r0 + rb
        pltpu.sync_copy(e3.at[0, pl.ds(r, 1)], idxr_v.at[pl.ds(0, 1)])
        pltpu.sync_copy(e3.at[1, pl.ds(r, 1)], idxc_v.at[pl.ds(0, 1)])
        pltpu.async_copy(dsi.at[idxr_v.at[0]], bufr.at[pl.ds(0, 128)], semr).wait()
        pltpu.async_copy(dsi.at[idxc_v.at[0]], bufc.at[pl.ds(0, 128)], semc).wait()
        pltpu.sync_copy(maps_in.at[pl.ds(r * 128, 128)], bufa.at[pl.ds(0, 128)])
        pltpu.sync_copy(maps_in.at[pl.ds(half_e + r * 128, 128)],
                        bufb.at[pl.ds(0, 128)])
        mul_loop((128 * D) // 16)
        pltpu.sync_copy(bufa.at[pl.ds(0, 128)], tril.at[pl.ds(r * 128, 128)])


@jax.jit
def kernel(maps, edge_index):
    half = edge_index.shape[1] // 2
    e3 = edge_index.reshape(2, (2 * half) // 128, 128)
    mesh = plsc.VectorSubcoreMesh(core_axis_name="c", subcore_axis_name="s")

    part = pl.kernel(
        _scatter_body,
        out_type=jax.ShapeDtypeStruct((NC, SIZE_P, D), jnp.float32),
        mesh=mesh,
        compiler_params=pltpu.CompilerParams(use_tc_tiling_on_sc=False, needs_layout_passes=False),
        scratch_types=[
            pltpu.VMEM((KC, 128), jnp.int32),
            pltpu.VMEM((CE, D), jnp.float32),
            pltpu.VMEM_SHARED((SIZE_P, D), jnp.float32),
        ],
    )(e3, maps)

    nrm = SIZE_P * D // 1024
    dsi_f, dn_f = pl.pallas_call(
        _norm_body,
        out_shape=[
            jax.ShapeDtypeStruct((nrm, 1024), jnp.float32),
            jax.ShapeDtypeStruct((nrm, 1024), jnp.float32),
        ],
    )(part.reshape(NC, nrm, 1024))

    dsi8 = jnp.concatenate(
        [dsi_f.reshape(SIZE_P, D),
         jnp.zeros((SIZE_P, 8 - D), jnp.float32)], axis=1)

    tril = pl.kernel(
        _gather_body,
        out_type=jax.ShapeDtypeStruct((half, D), jnp.float32),
        mesh=mesh,
        compiler_params=pltpu.CompilerParams(use_tc_tiling_on_sc=False, needs_layout_passes=False),
        scratch_types=[
            pltpu.VMEM((KCD, 128), jnp.int32),
            pltpu.VMEM((KCD, 128), jnp.int32),
            pltpu.VMEM((CED, 8), jnp.float32),
            pltpu.VMEM((CED, 8), jnp.float32),
            pltpu.VMEM((CED, D), jnp.float32),
            pltpu.VMEM((CED, D), jnp.float32),
            pltpu.SemaphoreType.DMA,
            pltpu.SemaphoreType.DMA,
        ],
    )(e3, maps, dsi8)

    return jnp.concatenate([dn_f.reshape(SIZE_P, D)[:NUM_NODES], tril], axis=0)


# speedup vs baseline: 15.5469x; 6.7059x over previous
import jax
import jax.numpy as jnp
from jax import lax
from jax.experimental import pallas as pl
from jax.experimental.pallas import tpu as pltpu
from jax.experimental.pallas import tpu_sc as plsc

NUM_NODES = 50000
D = 4
NC, NS = 2, 16
NW = NC * NS

RPT = 3136
SIZE_P = NS * RPT
NBLK = SIZE_P // 128

KC = 39
CE = KC * 128
KCD = 13
CED = KCD * 128

_SC_PARAMS = dict(
    compiler_params=pltpu.CompilerParams(
        use_tc_tiling_on_sc=False, needs_layout_passes=False),
)


def _idx16(i):
    p = i * 16 + lax.iota(jnp.int32, 16)
    e = lax.shift_right_logical(p, 2)
    c = jnp.bitwise_and(p, 3)
    br = jnp.bitwise_or(
        lax.shift_left(lax.shift_right_logical(e, 7), 2), c)
    bc = jnp.bitwise_and(e, 127)
    return e, c, br, bc


def _wid():
    return lax.axis_index("s") * NC + lax.axis_index("c")


def _scatter_body(e3, mv, part, idx_v, stg_v, vals_v, acc_sh):
    nrows = e3.shape[1]
    rb = nrows // NW
    rem = nrows - rb * NW
    cid = lax.axis_index("c")
    sid = lax.axis_index("s")
    wid = sid * NC + cid

    def zb(i, k):
        e, c, _, _ = _idx16(i)
        plsc.store_scatter(vals_v, [e, c], jnp.zeros((16,), jnp.float32))
        return k
    lax.fori_loop(0, (RPT * D) // 16, zb, 0)
    pltpu.sync_copy(vals_v.at[pl.ds(0, RPT)], acc_sh.at[pl.ds(sid * RPT, RPT)])
    plsc.subcore_barrier()

    r0 = wid * rb + jnp.minimum(wid, rem)

    def sq_chunk(nk):
        def body(i, k):
            e, c, br, bc = _idx16(i)
            x = plsc.load_gather(stg_v, [br, bc])
            plsc.store_scatter(vals_v, [e, c], x * x)
            return k
        lax.fori_loop(0, (nk * 128 * D) // 16, body, 0)

    def chunk(cc, carry):
        r = r0 + cc * KC
        pltpu.sync_copy(e3.at[0, pl.ds(r, KC)], idx_v)
        pltpu.sync_copy(mv.at[pl.ds(r * D, KC * D)], stg_v)
        sq_chunk(KC)
        for j in range(KC):
            pltpu.sync_copy(vals_v.at[pl.ds(j * 128, 128)],
                            acc_sh.at[idx_v.at[j]], add=True)
        return carry
    lax.fori_loop(0, rb // KC, chunk, 0)

    @pl.when(wid < rem)
    def _extra():
        r = r0 + rb
        pltpu.sync_copy(e3.at[0, pl.ds(r, 1)], idx_v.at[pl.ds(0, 1)])
        pltpu.sync_copy(mv.at[pl.ds(r * D, D)], stg_v.at[pl.ds(0, D)])
        sq_chunk(1)
        pltpu.sync_copy(vals_v.at[pl.ds(0, 128)],
                        acc_sh.at[idx_v.at[0]], add=True)

    plsc.subcore_barrier()
    pltpu.sync_copy(acc_sh.at[pl.ds(sid * RPT, RPT)], vals_v.at[pl.ds(0, RPT)])
    pltpu.sync_copy(vals_v.at[pl.ds(0, RPT)], part.at[cid, pl.ds(sid * RPT, RPT)])


def _rsqrt_newton(x):
    xi = plsc.bitcast(x, jnp.int32)
    y = plsc.bitcast(
        jnp.int32(0x5F3759DF) - lax.shift_right_logical(xi, 1), jnp.float32)
    hx = x * 0.5
    for _ in range(3):
        y = y * (1.5 - hx * y * y)
    return y


def _norm_body(part, dsi8, dnv, p0_v, p1_v, dsi_v, dn_v):
    bpt = NBLK // NW
    brem = NBLK - bpt * NW
    wid = _wid()
    b0 = wid * bpt + jnp.minimum(wid, brem)
    nb = bpt + jnp.where(wid < brem, 1, 0)
    n0 = b0 * 128
    nn = nb * 128

    pltpu.sync_copy(part.at[0, pl.ds(n0, nn)], p0_v.at[pl.ds(0, nn)])
    pltpu.sync_copy(part.at[1, pl.ds(n0, nn)], p1_v.at[pl.ds(0, nn)])

    def body(i, k):
        e, c, br, bc = _idx16(i)
        d = (plsc.load_gather(p0_v, [e, c])
             + plsc.load_gather(p1_v, [e, c]))
        dp1 = d + 1.0
        plsc.store_scatter(dsi_v, [e, c], _rsqrt_newton(dp1))
        plsc.store_scatter(dn_v, [br, bc], d / dp1)
        return k
    lax.fori_loop(0, (nn * D) // 16, body, 0)

    pltpu.sync_copy(dsi_v.at[pl.ds(0, nn)], dsi8.at[pl.ds(n0, nn)])
    pltpu.sync_copy(dn_v.at[pl.ds(0, nb * D)], dnv.at[pl.ds(b0 * D, nb * D)])


def _gather_body(e3, mv, dsi8, trilv, idxr_v, idxc_v, bufr, bufc, stgl, stgr,
                 out_v, semr, semc):
    nrows = e3.shape[1] // 2
    rb = nrows // NW
    rem = nrows - rb * NW
    wid = _wid()
    half_mv = mv.shape[0] // 2
    r0 = wid * rb + jnp.minimum(wid, rem)

    def mul_chunk(nk):
        def body(i, k):
            e, c, br, bc = _idx16(i)
            a = plsc.load_gather(stgl, [br, bc])
            b = plsc.load_gather(stgr, [br, bc])
            rr = plsc.load_gather(bufr, [e, c])
            cc2 = plsc.load_gather(bufc, [e, c])
            plsc.store_scatter(out_v, [br, bc], -(a * b) * rr * cc2)
            return k
        lax.fori_loop(0, (nk * 128 * D) // 16, body, 0)

    def chunk(cc, carry):
        r = r0 + cc * KCD
        pltpu.sync_copy(e3.at[0, pl.ds(r, KCD)], idxr_v)
        pltpu.sync_copy(e3.at[1, pl.ds(r, KCD)], idxc_v)
        cps = []
        for j in range(KCD):
            cps.append(pltpu.async_copy(
                dsi8.at[idxr_v.at[j]], bufr.at[pl.ds(j * 128, 128)], semr))
            cps.append(pltpu.async_copy(
                dsi8.at[idxc_v.at[j]], bufc.at[pl.ds(j * 128, 128)], semc))
        pltpu.sync_copy(mv.at[pl.ds(r * D, KCD * D)], stgl)
        pltpu.sync_copy(mv.at[pl.ds(half_mv + r * D, KCD * D)], stgr)
        for cp in cps:
            cp.wait()
        mul_chunk(KCD)
        pltpu.sync_copy(out_v, trilv.at[pl.ds(r * D, KCD * D)])
        return carry
    lax.fori_loop(0, rb // KCD, chunk, 0)

    @pl.when(wid < rem)
    def _extra():
        r = r0 + rb
        pltpu.sync_copy(e3.at[0, pl.ds(r, 1)], idxr_v.at[pl.ds(0, 1)])
        pltpu.sync_copy(e3.at[1, pl.ds(r, 1)], idxc_v.at[pl.ds(0, 1)])
        pltpu.async_copy(dsi8.at[idxr_v.at[0]], bufr.at[pl.ds(0, 128)],
                         semr).wait()
        pltpu.async_copy(dsi8.at[idxc_v.at[0]], bufc.at[pl.ds(0, 128)],
                         semc).wait()
        pltpu.sync_copy(mv.at[pl.ds(r * D, D)], stgl.at[pl.ds(0, D)])
        pltpu.sync_copy(mv.at[pl.ds(half_mv + r * D, D)], stgr.at[pl.ds(0, D)])
        mul_chunk(1)
        pltpu.sync_copy(out_v.at[pl.ds(0, D)], trilv.at[pl.ds(r * D, D)])


@jax.jit
def kernel(maps, edge_index):
    half = edge_index.shape[1] // 2
    nblk_e = (2 * half) // 128
    e3 = edge_index.reshape(2, nblk_e, 128)
    mv = maps.reshape(nblk_e, 128, D).transpose(0, 2, 1).reshape(nblk_e * D, 128)
    mesh = plsc.VectorSubcoreMesh(core_axis_name="c", subcore_axis_name="s")

    part = pl.kernel(
        _scatter_body,
        out_type=jax.ShapeDtypeStruct((NC, SIZE_P, D), jnp.float32),
        mesh=mesh,
        scratch_types=[
            pltpu.VMEM((KC, 128), jnp.int32),
            pltpu.VMEM((KC * D, 128), jnp.float32),
            pltpu.VMEM((CE, D), jnp.float32),
            pltpu.VMEM_SHARED((SIZE_P, D), jnp.float32),
        ],
        **_SC_PARAMS,
    )(e3, mv)

    bmax = NBLK // NW + 1
    dsi8, dnv = pl.kernel(
        _norm_body,
        out_type=[
            jax.ShapeDtypeStruct((SIZE_P, 8), jnp.float32),
            jax.ShapeDtypeStruct((NBLK * D, 128), jnp.float32),
        ],
        mesh=mesh,
        scratch_types=[
            pltpu.VMEM((bmax * 128, D), jnp.float32),
            pltpu.VMEM((bmax * 128, D), jnp.float32),
            pltpu.VMEM((bmax * 128, 8), jnp.float32),
            pltpu.VMEM((bmax * D, 128), jnp.float32),
        ],
        **_SC_PARAMS,
    )(part)

    trilv = pl.kernel(
        _gather_body,
        out_type=jax.ShapeDtypeStruct((half // 128 * D, 128), jnp.float32),
        mesh=mesh,
        scratch_types=[
            pltpu.VMEM((KCD, 128), jnp.int32),
            pltpu.VMEM((KCD, 128), jnp.int32),
            pltpu.VMEM((CED, 8), jnp.float32),
            pltpu.VMEM((CED, 8), jnp.float32),
            pltpu.VMEM((KCD * D, 128), jnp.float32),
            pltpu.VMEM((KCD * D, 128), jnp.float32),
            pltpu.VMEM((KCD * D, 128), jnp.float32),
            pltpu.SemaphoreType.DMA,
            pltpu.SemaphoreType.DMA,
        ],
        **_SC_PARAMS,
    )(e3, mv, dsi8)

    dn = dnv.reshape(NBLK, D, 128).transpose(0, 2, 1).reshape(SIZE_P, D)
    tril = trilv.reshape(half // 128, D, 128).transpose(0, 2, 1).reshape(half, D)
    return jnp.concatenate([dn[:NUM_NODES], tril], axis=0)


# speedup vs baseline: 22.0258x; 1.4167x over previous
import jax
import jax.numpy as jnp
from jax import lax
from jax.experimental import pallas as pl
from jax.experimental.pallas import tpu as pltpu
from jax.experimental.pallas import tpu_sc as plsc

NUM_NODES = 50000
D = 4
NC, NS = 2, 16
NW = NC * NS

RPT = 3136
SIZE_P = NS * RPT
NBLK = SIZE_P // 128

KC = 15
CE = KC * 128
NCH_B = 26
KCD = 13
CED = KCD * 128
NCH_D = 15

_SC_PARAMS = dict(
    compiler_params=pltpu.CompilerParams(
        use_tc_tiling_on_sc=False, needs_layout_passes=False),
)


def _wid():
    return lax.axis_index("s") * NC + lax.axis_index("c")


def _ec16(i):
    q = i * 16 + lax.iota(jnp.int32, 16)
    e = jnp.bitwise_or(
        lax.shift_left(lax.shift_right_logical(q, 9), 7),
        jnp.bitwise_and(q, 127))
    c = jnp.bitwise_and(lax.shift_right_logical(q, 7), 3)
    return e, c


def _scatter_body(e3, mv16, part, idx0, idx1, stg16, vals0, vals1, acc_sh,
                  sem_st, sem_s0, sem_s1):
    nrows = e3.shape[1]
    rb = nrows // NW
    rem = nrows - rb * NW
    cid = lax.axis_index("c")
    sid = lax.axis_index("s")
    wid = sid * NC + cid
    r0 = wid * rb + jnp.minimum(wid, rem)

    def zb(i, k):
        e, c = _ec16(i)
        plsc.store_scatter(vals0, [e, c], jnp.zeros((16,), jnp.float32))
        return k
    lax.fori_loop(0, (RPT * D) // 16, zb, 0)
    pltpu.sync_copy(vals0.at[pl.ds(0, RPT)], acc_sh.at[pl.ds(sid * RPT, RPT)])
    plsc.subcore_barrier()

    def sq(vals_b, nk):
        def body(i, k):
            e, c = _ec16(i)
            x = stg16[i]
            plsc.store_scatter(vals_b, [e, c], x * x)
            return k
        lax.fori_loop(0, (nk * 128 * D) // 16, body, 0)

    def stage(r, idx_b):
        pltpu.async_copy(e3.at[0, pl.ds(r, KC)], idx_b, sem_st)
        pltpu.async_copy(mv16.at[pl.ds(r * D * 8, KC * D * 8)], stg16, sem_st)

    def wait_stage(idx_b):
        pltpu.make_async_copy(e3.at[0, pl.ds(0, KC)], idx_b, sem_st).wait()
        pltpu.make_async_copy(
            mv16.at[pl.ds(0, KC * D * 8)], stg16, sem_st).wait()

    def fire(vals_b, idx_b, sem_b):
        for j in range(KC):
            pltpu.async_copy(vals_b.at[pl.ds(j * 128, 128)],
                             acc_sh.at[idx_b.at[j]], sem_b, add=True)

    def drain(vals_b, idx_b, sem_b):
        for j in range(KC):
            pltpu.make_async_copy(vals_b.at[pl.ds(j * 128, 128)],
                                  acc_sh.at[idx_b.at[j]], sem_b).wait()

    stage(r0, idx0)

    def pair(t, carry):
        k0 = 2 * t
        wait_stage(idx0)
        sq(vals0, KC)

        @pl.when(t > 0)
        def _():
            drain(vals1, idx1, sem_s1)
        fire(vals0, idx0, sem_s0)
        stage(r0 + (k0 + 1) * KC, idx1)
        wait_stage(idx1)
        sq(vals1, KC)
        drain(vals0, idx0, sem_s0)

        fire(vals1, idx1, sem_s1)

        @pl.when(t < NCH_B // 2 - 1)
        def _():
            stage(r0 + (k0 + 2) * KC, idx0)
        return carry
    lax.fori_loop(0, NCH_B // 2, pair, 0)
    drain(vals1, idx1, sem_s1)

    @pl.when(wid < rem)
    def _extra():
        r = r0 + rb
        pltpu.sync_copy(e3.at[0, pl.ds(r, 1)], idx0.at[pl.ds(0, 1)])
        pltpu.sync_copy(mv16.at[pl.ds(r * D * 8, D * 8)],
                        stg16.at[pl.ds(0, D * 8)])
        sq(vals0, 1)
        pltpu.sync_copy(vals0.at[pl.ds(0, 128)],
                        acc_sh.at[idx0.at[0]], add=True)

    plsc.subcore_barrier()
    pltpu.sync_copy(acc_sh.at[pl.ds(sid * RPT, RPT)], vals0.at[pl.ds(0, RPT)])
    pltpu.sync_copy(vals0.at[pl.ds(0, RPT)],
                    part.at[cid, pl.ds(sid * RPT, RPT)])


def _rsqrt_newton(x):
    xi = plsc.bitcast(x, jnp.int32)
    y = plsc.bitcast(
        jnp.int32(0x5F3759DF) - lax.shift_right_logical(xi, 1), jnp.float32)
    hx = x * 0.5
    for _ in range(3):
        y = y * (1.5 - hx * y * y)
    return y


def _norm_body(part, dsi8, dnv, p0_v, p1_v, dsi_v, dn_v):
    bpt = NBLK // NW
    brem = NBLK - bpt * NW
    wid = _wid()
    b0 = wid * bpt + jnp.minimum(wid, brem)
    nb = bpt + jnp.where(wid < brem, 1, 0)
    n0 = b0 * 128
    nn = nb * 128

    pltpu.sync_copy(part.at[0, pl.ds(n0, nn)], p0_v.at[pl.ds(0, nn)])
    pltpu.sync_copy(part.at[1, pl.ds(n0, nn)], p1_v.at[pl.ds(0, nn)])

    def body(i, k):
        e, c = _ec16(i)
        q = i * 16 + lax.iota(jnp.int32, 16)
        br = lax.shift_right_logical(q, 7)
        bc = jnp.bitwise_and(q, 127)
        d = (plsc.load_gather(p0_v, [e, c])
             + plsc.load_gather(p1_v, [e, c]))
        dp1 = d + 1.0
        plsc.store_scatter(dsi_v, [e, c], _rsqrt_newton(dp1))
        plsc.store_scatter(dn_v, [br, bc], d / dp1)
        return k
    lax.fori_loop(0, (nn * D) // 16, body, 0)

    pltpu.sync_copy(dsi_v.at[pl.ds(0, nn)], dsi8.at[pl.ds(n0, nn)])
    pltpu.sync_copy(dn_v.at[pl.ds(0, nb * D)], dnv.at[pl.ds(b0 * D, nb * D)])


def _gather_body(e3, mv16, dsi8, trilv, idx0, idx1, bufr0, bufc0, bufr1,
                 bufc1, stgl0, stgr0, stgl1, stgr1, out0, out1,
                 sem_ix, sem_g0, sem_g1, sem_st0, sem_st1, sem_w):
    nrows = e3.shape[1] // 2
    rb = nrows // NW
    rem = nrows - rb * NW
    wid = _wid()
    half16 = mv16.shape[0] // 2
    r0 = wid * rb + jnp.minimum(wid, rem)

    idxs = (idx0, idx1)
    bufrs = (bufr0, bufr1)
    bufcs = (bufc0, bufc1)
    stgls = (stgl0, stgl1)
    stgrs = (stgr0, stgr1)
    outs = (out0, out1)
    sem_gs = (sem_g0, sem_g1)
    sem_sts = (sem_st0, sem_st1)

    def stage_idx(r, b):
        pltpu.async_copy(e3.at[0, pl.ds(r, KCD)], idxs[b].at[0], sem_ix)
        pltpu.async_copy(e3.at[1, pl.ds(r, KCD)], idxs[b].at[1], sem_ix)

    def wait_idx(b):
        pltpu.make_async_copy(
            e3.at[0, pl.ds(0, KCD)], idxs[b].at[0], sem_ix).wait()
        pltpu.make_async_copy(
            e3.at[1, pl.ds(0, KCD)], idxs[b].at[1], sem_ix).wait()

    def fire_gather(b):
        for j in range(KCD):
            pltpu.async_copy(dsi8.at[idxs[b].at[0].at[j]],
                             bufrs[b].at[pl.ds(j * 128, 128)], sem_gs[b])
            pltpu.async_copy(dsi8.at[idxs[b].at[1].at[j]],
                             bufcs[b].at[pl.ds(j * 128, 128)], sem_gs[b])

    def drain_gather(b):
        for j in range(KCD):
            pltpu.make_async_copy(dsi8.at[idxs[b].at[0].at[j]],
                                  bufrs[b].at[pl.ds(j * 128, 128)],
                                  sem_gs[b]).wait()
            pltpu.make_async_copy(dsi8.at[idxs[b].at[1].at[j]],
                                  bufcs[b].at[pl.ds(j * 128, 128)],
                                  sem_gs[b]).wait()

    def stage_data(r, b):
        pltpu.async_copy(mv16.at[pl.ds(r * D * 8, KCD * D * 8)],
                         stgls[b], sem_sts[b])
        pltpu.async_copy(mv16.at[pl.ds(half16 + r * D * 8, KCD * D * 8)],
                         stgrs[b], sem_sts[b])

    def wait_data(b):
        pltpu.make_async_copy(
            mv16.at[pl.ds(0, KCD * D * 8)], stgls[b], sem_sts[b]).wait()
        pltpu.make_async_copy(
            mv16.at[pl.ds(0, KCD * D * 8)], stgrs[b], sem_sts[b]).wait()

    def mul(b, nk):
        def body(i, k):
            e, c = _ec16(i)
            a = stgls[b][i]
            bb = stgrs[b][i]
            rr = plsc.load_gather(bufrs[b], [e, c])
            cc2 = plsc.load_gather(bufcs[b], [e, c])
            outs[b][i] = -(a * bb) * rr * cc2
            return k
        lax.fori_loop(0, (nk * 128 * D) // 16, body, 0)

    def wb_start(r, b):
        pltpu.async_copy(outs[b], trilv.at[pl.ds(r * D * 8, KCD * D * 8)],
                         sem_w)

    def wb_drain(b):
        pltpu.make_async_copy(
            outs[b], trilv.at[pl.ds(0, KCD * D * 8)], sem_w).wait()

    pltpu.sync_copy(e3.at[0, pl.ds(r0, KCD)], idx0.at[0])
    pltpu.sync_copy(e3.at[1, pl.ds(r0, KCD)], idx0.at[1])
    fire_gather(0)
    stage_data(r0, 0)
    stage_idx(r0 + KCD, 1)

    def half(t, k, b):
        wait_idx(1 - b)
        fire_gather(1 - b)
        stage_data(r0 + (k + 1) * KCD, 1 - b)
        drain_gather(b)
        wait_data(b)

        @pl.when(k + 2 < NCH_D)
        def _():
            stage_idx(r0 + (k + 2) * KCD, b)

        @pl.when(t > 0)
        def _():
            wb_drain(b)
        mul(b, KCD)
        wb_start(r0 + k * KCD, b)

    def pair(t, carry):
        half(t, 2 * t, 0)
        half(t, 2 * t + 1, 1)
        return carry
    lax.fori_loop(0, NCH_D // 2, pair, 0)

    k_last = NCH_D - 1
    drain_gather(0)
    wait_data(0)
    wb_drain(0)
    mul(0, KCD)
    wb_drain(1)
    wb_start(r0 + k_last * KCD, 0)

    @pl.when(wid < rem)
    def _extra():
        r = r0 + rb
        pltpu.sync_copy(e3.at[0, pl.ds(r, 1)], idx1.at[0].at[pl.ds(0, 1)])
        pltpu.sync_copy(e3.at[1, pl.ds(r, 1)], idx1.at[1].at[pl.ds(0, 1)])
        pltpu.async_copy(dsi8.at[idx1.at[0].at[0]],
                         bufr1.at[pl.ds(0, 128)], sem_g1).wait()
        pltpu.async_copy(dsi8.at[idx1.at[1].at[0]],
                         bufc1.at[pl.ds(0, 128)], sem_g1).wait()
        pltpu.sync_copy(mv16.at[pl.ds(r * D * 8, D * 8)],
                        stgl1.at[pl.ds(0, D * 8)])
        pltpu.sync_copy(mv16.at[pl.ds(half16 + r * D * 8, D * 8)],
                        stgr1.at[pl.ds(0, D * 8)])
        mul(1, 1)
        pltpu.sync_copy(out1.at[pl.ds(0, D * 8)],
                        trilv.at[pl.ds(r * D * 8, D * 8)])

    wb_drain(0)


@jax.jit
def kernel(maps, edge_index):
    half = edge_index.shape[1] // 2
    nblk_e = (2 * half) // 128
    e3 = edge_index.reshape(2, nblk_e, 128)
    mv16 = (maps.reshape(nblk_e, 128, D).transpose(0, 2, 1)
            .reshape(nblk_e * D * 8, 16))
    mesh = plsc.VectorSubcoreMesh(core_axis_name="c", subcore_axis_name="s")

    part = pl.kernel(
        _scatter_body,
        out_type=jax.ShapeDtypeStruct((NC, SIZE_P, D), jnp.float32),
        mesh=mesh,
        scratch_types=[
            pltpu.VMEM((KC, 128), jnp.int32),
            pltpu.VMEM((KC, 128), jnp.int32),
            pltpu.VMEM((KC * D * 8, 16), jnp.float32),
            pltpu.VMEM((CE, D), jnp.float32),
            pltpu.VMEM((CE, D), jnp.float32),
            pltpu.VMEM_SHARED((SIZE_P, D), jnp.float32),
            pltpu.SemaphoreType.DMA,
            pltpu.SemaphoreType.DMA,
            pltpu.SemaphoreType.DMA,
        ],
        **_SC_PARAMS,
    )(e3, mv16)

    bmax = NBLK // NW + 1
    dsi8, dnv = pl.kernel(
        _norm_body,
        out_type=[
            jax.ShapeDtypeStruct((SIZE_P, 8), jnp.float32),
            jax.ShapeDtypeStruct((NBLK * D, 128), jnp.float32),
        ],
        mesh=mesh,
        scratch_types=[
            pltpu.VMEM((bmax * 128, D), jnp.float32),
            pltpu.VMEM((bmax * 128, D), jnp.float32),
            pltpu.VMEM((bmax * 128, 8), jnp.float32),
            pltpu.VMEM((bmax * D, 128), jnp.float32),
        ],
        **_SC_PARAMS,
    )(part)

    trilv = pl.kernel(
        _gather_body,
        out_type=jax.ShapeDtypeStruct((half * D // 16, 16), jnp.float32),
        mesh=mesh,
        scratch_types=[
            pltpu.VMEM((2, KCD, 128), jnp.int32),
            pltpu.VMEM((2, KCD, 128), jnp.int32),
            pltpu.VMEM((CED, 8), jnp.float32),
            pltpu.VMEM((CED, 8), jnp.float32),
            pltpu.VMEM((CED, 8), jnp.float32),
            pltpu.VMEM((CED, 8), jnp.float32),
            pltpu.VMEM((KCD * D * 8, 16), jnp.float32),
            pltpu.VMEM((KCD * D * 8, 16), jnp.float32),
            pltpu.VMEM((KCD * D * 8, 16), jnp.float32),
            pltpu.VMEM((KCD * D * 8, 16), jnp.float32),
            pltpu.VMEM((KCD * D * 8, 16), jnp.float32),
            pltpu.VMEM((KCD * D * 8, 16), jnp.float32),
            pltpu.SemaphoreType.DMA,
            pltpu.SemaphoreType.DMA,
            pltpu.SemaphoreType.DMA,
            pltpu.SemaphoreType.DMA,
            pltpu.SemaphoreType.DMA,
            pltpu.SemaphoreType.DMA,
        ],
        **_SC_PARAMS,
    )(e3, mv16, dsi8)

    dn = dnv.reshape(NBLK, D, 128).transpose(0, 2, 1).reshape(SIZE_P, D)
    tril = (trilv.reshape(half // 128, D, 128).transpose(0, 2, 1)
            .reshape(half, D))
    return jnp.concatenate([dn[:NUM_NODES], tril], axis=0)


# speedup vs baseline: 22.1776x; 1.0069x over previous
import jax
import jax.numpy as jnp
from jax import lax
from jax.experimental import pallas as pl
from jax.experimental.pallas import tpu as pltpu
from jax.experimental.pallas import tpu_sc as plsc

NUM_NODES = 50000
D = 4
NC, NS = 2, 16
NW = NC * NS

RPT = 3136
SIZE_P = NS * RPT
NBLK = SIZE_P // 128

CE = 1920
NCH_B = 26
EPT_B = 49984
CED = 1664
NCH_D = 15
EPT_D = 24992

_SC_PARAMS = dict(
    compiler_params=pltpu.CompilerParams(
        use_tc_tiling_on_sc=False, needs_layout_passes=False),
)


def _wid():
    return lax.axis_index("s") * NC + lax.axis_index("c")


def _ec16(i):
    q = i * 16 + lax.iota(jnp.int32, 16)
    e = jnp.bitwise_or(
        lax.shift_left(lax.shift_right_logical(q, 9), 7),
        jnp.bitwise_and(q, 127))
    c = jnp.bitwise_and(lax.shift_right_logical(q, 7), 3)
    return e, c


def _scatter_body(e2, mv16, part, idx0, idx1, stg16, vals0, vals1, tidx,
                  tstg, tvals, acc_sh, sem_st, sem_s0, sem_s1):
    cid = lax.axis_index("c")
    sid = lax.axis_index("s")
    wid = sid * NC + cid
    base = wid * EPT_B + 32 * jnp.minimum(wid, NW // 2)

    def zb(i, k):
        e, c = _ec16(i)
        plsc.store_scatter(vals0, [e, c], jnp.zeros((16,), jnp.float32))
        return k
    lax.fori_loop(0, (CE * D) // 16, zb, 0)
    pltpu.sync_copy(vals0, acc_sh.at[pl.ds(sid * RPT, CE)])
    pltpu.sync_copy(vals0.at[pl.ds(0, RPT - CE)],
                    acc_sh.at[pl.ds(sid * RPT + CE, RPT - CE)])
    plsc.subcore_barrier()

    def sq(vals_b, ne):
        def body(i, k):
            e, c = _ec16(i)
            x = stg16[i]
            plsc.store_scatter(vals_b, [e, c], x * x)
            return k
        lax.fori_loop(0, (ne * D) // 16, body, 0)

    def stage(e0, idx_b):
        pltpu.async_copy(e2.at[0, pl.ds(e0, CE)], idx_b, sem_st)
        pltpu.async_copy(mv16.at[pl.ds(e0 // 4, CE // 4)], stg16, sem_st)

    def wait_stage(idx_b):
        pltpu.make_async_copy(e2.at[0, pl.ds(0, CE)], idx_b, sem_st).wait()
        pltpu.make_async_copy(
            mv16.at[pl.ds(0, CE // 4)], stg16, sem_st).wait()

    stage(base, idx0)

    def pair(t, carry):
        k0 = 2 * t
        wait_stage(idx0)
        sq(vals0, CE)

        @pl.when(t > 0)
        def _():
            pltpu.make_async_copy(vals1, acc_sh.at[idx1], sem_s1).wait()
        pltpu.async_copy(vals0, acc_sh.at[idx0], sem_s0, add=True)
        stage(base + (k0 + 1) * CE, idx1)

        wait_stage(idx1)
        sq(vals1, CE)
        pltpu.make_async_copy(vals0, acc_sh.at[idx0], sem_s0).wait()
        pltpu.async_copy(vals1, acc_sh.at[idx1], sem_s1, add=True)

        @pl.when(t < NCH_B // 2 - 1)
        def _():
            stage(base + (k0 + 2) * CE, idx0)
        return carry
    lax.fori_loop(0, NCH_B // 2, pair, 0)
    pltpu.make_async_copy(vals1, acc_sh.at[idx1], sem_s1).wait()

    e0t = base + NCH_B * CE

    def tail(ne):
        pltpu.sync_copy(e2.at[0, pl.ds(e0t, ne)], tidx.at[pl.ds(0, ne)])
        pltpu.sync_copy(mv16.at[pl.ds(e0t // 4, ne // 4)],
                        tstg.at[pl.ds(0, ne // 4)])

        def body(i, k):
            q = i * 16 + lax.iota(jnp.int32, 16)
            e = lax.shift_right_logical(q, 2)
            c = jnp.bitwise_and(q, 3)
            x = tstg[i]
            plsc.store_scatter(tvals, [e, c], x * x)
            return k
        lax.fori_loop(0, (ne * D) // 16, body, 0)
        pltpu.sync_copy(tvals.at[pl.ds(0, ne)],
                        acc_sh.at[tidx.at[pl.ds(0, ne)]], add=True)

    @pl.when(wid < NW // 2)
    def _():
        tail(96)

    @pl.when(wid >= NW // 2)
    def _():
        tail(64)

    plsc.subcore_barrier()
    pltpu.sync_copy(acc_sh.at[pl.ds(sid * RPT, CE)], vals0)
    pltpu.sync_copy(vals0, part.at[cid, pl.ds(sid * RPT, CE)])
    pltpu.sync_copy(acc_sh.at[pl.ds(sid * RPT + CE, RPT - CE)],
                    vals1.at[pl.ds(0, RPT - CE)])
    pltpu.sync_copy(vals1.at[pl.ds(0, RPT - CE)],
                    part.at[cid, pl.ds(sid * RPT + CE, RPT - CE)])


def _rsqrt_newton(x):
    xi = plsc.bitcast(x, jnp.int32)
    y = plsc.bitcast(
        jnp.int32(0x5F3759DF) - lax.shift_right_logical(xi, 1), jnp.float32)
    hx = x * 0.5
    for _ in range(3):
        y = y * (1.5 - hx * y * y)
    return y


def _norm_body(part, dsi8, dnv, p0_v, p1_v, dsi_v, dn_v):
    bpt = NBLK // NW
    brem = NBLK - bpt * NW
    wid = _wid()
    b0 = wid * bpt + jnp.minimum(wid, brem)
    nb = bpt + jnp.where(wid < brem, 1, 0)
    n0 = b0 * 128
    nn = nb * 128

    pltpu.sync_copy(part.at[0, pl.ds(n0, nn)], p0_v.at[pl.ds(0, nn)])
    pltpu.sync_copy(part.at[1, pl.ds(n0, nn)], p1_v.at[pl.ds(0, nn)])

    def body(i, k):
        e, c = _ec16(i)
        q = i * 16 + lax.iota(jnp.int32, 16)
        br = lax.shift_right_logical(q, 7)
        bc = jnp.bitwise_and(q, 127)
        d = (plsc.load_gather(p0_v, [e, c])
             + plsc.load_gather(p1_v, [e, c]))
        dp1 = d + 1.0
        plsc.store_scatter(dsi_v, [e, c], _rsqrt_newton(dp1))
        plsc.store_scatter(dn_v, [br, bc], d / dp1)
        return k
    lax.fori_loop(0, (nn * D) // 16, body, 0)

    pltpu.sync_copy(dsi_v.at[pl.ds(0, nn)], dsi8.at[pl.ds(n0, nn)])
    pltpu.sync_copy(dn_v.at[pl.ds(0, nb * D)], dnv.at[pl.ds(b0 * D, nb * D)])


def _gather_body(e2, mv16, dsi8, trilv, idx0, idx1, bufr0, bufc0, bufr1,
                 bufc1, stgl0, stgr0, stgl1, stgr1, out0, out1, tidx,
                 sem_ix, sem_g0, sem_g1, sem_st0, sem_st1, sem_w):
    wid = _wid()
    half_e = e2.shape[1] // 2
    base = wid * EPT_D + 32 * jnp.minimum(wid, 8)

    idxs = (idx0, idx1)
    bufrs = (bufr0, bufr1)
    bufcs = (bufc0, bufc1)
    stgls = (stgl0, stgl1)
    stgrs = (stgr0, stgr1)
    outs = (out0, out1)
    sem_gs = (sem_g0, sem_g1)
    sem_sts = (sem_st0, sem_st1)

    def stage_idx(e0, b):
        pltpu.async_copy(e2.at[0, pl.ds(e0, CED)], idxs[b].at[0], sem_ix)
        pltpu.async_copy(e2.at[1, pl.ds(e0, CED)], idxs[b].at[1], sem_ix)

    def wait_idx(b):
        pltpu.make_async_copy(
            e2.at[0, pl.ds(0, CED)], idxs[b].at[0], sem_ix).wait()
        pltpu.make_async_copy(
            e2.at[1, pl.ds(0, CED)], idxs[b].at[1], sem_ix).wait()

    def fire_gather(b):
        pltpu.async_copy(dsi8.at[idxs[b].at[0]], bufrs[b], sem_gs[b])
        pltpu.async_copy(dsi8.at[idxs[b].at[1]], bufcs[b], sem_gs[b])

    def drain_gather(b):
        pltpu.make_async_copy(
            dsi8.at[idxs[b].at[0]], bufrs[b], sem_gs[b]).wait()
        pltpu.make_async_copy(
            dsi8.at[idxs[b].at[1]], bufcs[b], sem_gs[b]).wait()

    def stage_data(e0, b):
        pltpu.async_copy(mv16.at[pl.ds(e0 // 4, CED // 4)],
                         stgls[b], sem_sts[b])
        pltpu.async_copy(mv16.at[pl.ds((half_e + e0) // 4, CED // 4)],
                         stgrs[b], sem_sts[b])

    def wait_data(b):
        pltpu.make_async_copy(
            mv16.at[pl.ds(0, CED // 4)], stgls[b], sem_sts[b]).wait()
        pltpu.make_async_copy(
            mv16.at[pl.ds(0, CED // 4)], stgrs[b], sem_sts[b]).wait()

    def mul(b, ne):
        def body(i, k):
            e, c = _ec16(i)
            a = stgls[b][i]
            bb = stgrs[b][i]
            rr = plsc.load_gather(bufrs[b], [e, c])
            cc2 = plsc.load_gather(bufcs[b], [e, c])
            outs[b][i] = -(a * bb) * rr * cc2
            return k
        lax.fori_loop(0, (ne * D) // 16, body, 0)

    def wb_start(e0, b):
        pltpu.async_copy(outs[b], trilv.at[pl.ds(e0 // 4, CED // 4)], sem_w)

    def wb_drain(b):
        pltpu.make_async_copy(
            outs[b], trilv.at[pl.ds(0, CED // 4)], sem_w).wait()

    pltpu.sync_copy(e2.at[0, pl.ds(base, CED)], idx0.at[0])
    pltpu.sync_copy(e2.at[1, pl.ds(base, CED)], idx0.at[1])
    fire_gather(0)
    stage_data(base, 0)
    stage_idx(base + CED, 1)

    def half(t, k, b):
        wait_idx(1 - b)
        fire_gather(1 - b)
        stage_data(base + (k + 1) * CED, 1 - b)
        drain_gather(b)
        wait_data(b)

        @pl.when(k + 2 < NCH_D)
        def _():
            stage_idx(base + (k + 2) * CED, b)

        @pl.when(t > 0)
        def _():
            wb_drain(b)
        mul(b, CED)
        wb_start(base + k * CED, b)

    def pair(t, carry):
        half(t, 2 * t, 0)
        half(t, 2 * t + 1, 1)
        return carry
    lax.fori_loop(0, NCH_D // 2, pair, 0)

    drain_gather(0)
    wait_data(0)
    wb_drain(0)
    mul(0, CED)
    wb_drain(1)
    wb_start(base + (NCH_D - 1) * CED, 0)

    e0t = base + NCH_D * CED

    def tail(ne):
        pltpu.sync_copy(e2.at[0, pl.ds(e0t, ne)], tidx.at[0].at[pl.ds(0, ne)])
        pltpu.sync_copy(e2.at[1, pl.ds(e0t, ne)], tidx.at[1].at[pl.ds(0, ne)])
        pltpu.async_copy(dsi8.at[tidx.at[0].at[pl.ds(0, ne)]],
                         bufr1.at[pl.ds(0, ne)], sem_g1).wait()
        pltpu.async_copy(dsi8.at[tidx.at[1].at[pl.ds(0, ne)]],
                         bufc1.at[pl.ds(0, ne)], sem_g1).wait()
        pltpu.sync_copy(mv16.at[pl.ds(e0t // 4, ne // 4)],
                        stgl1.at[pl.ds(0, ne // 4)])
        pltpu.sync_copy(mv16.at[pl.ds((half_e + e0t) // 4, ne // 4)],
                        stgr1.at[pl.ds(0, ne // 4)])

        def body(i, k):
            q = i * 16 + lax.iota(jnp.int32, 16)
            e = lax.shift_right_logical(q, 2)
            c = jnp.bitwise_and(q, 3)
            a = stgl1[i]
            bb = stgr1[i]
            rr = plsc.load_gather(bufr1, [e, c])
            cc2 = plsc.load_gather(bufc1, [e, c])
            out1[i] = -(a * bb) * rr * cc2
            return k
        lax.fori_loop(0, (ne * D) // 16, body, 0)
        pltpu.sync_copy(out1.at[pl.ds(0, ne // 4)],
                        trilv.at[pl.ds(e0t // 4, ne // 4)])

    @pl.when(wid < 8)
    def _():
        tail(64)

    @pl.when(wid >= 8)
    def _():
        tail(32)

    wb_drain(0)


@jax.jit
def kernel(maps, edge_index):
    half = edge_index.shape[1] // 2
    nblk_e = (2 * half) // 128
    mv16 = (maps.reshape(nblk_e, 128, D).transpose(0, 2, 1)
            .reshape(nblk_e * D * 8, 16))
    mesh = plsc.VectorSubcoreMesh(core_axis_name="c", subcore_axis_name="s")

    part = pl.kernel(
        _scatter_body,
        out_type=jax.ShapeDtypeStruct((NC, SIZE_P, D), jnp.float32),
        mesh=mesh,
        scratch_types=[
            pltpu.VMEM((CE,), jnp.int32),
            pltpu.VMEM((CE,), jnp.int32),
            pltpu.VMEM((CE // 4, 16), jnp.float32),
            pltpu.VMEM((CE, D), jnp.float32),
            pltpu.VMEM((CE, D), jnp.float32),
            pltpu.VMEM((96,), jnp.int32),
            pltpu.VMEM((24, 16), jnp.float32),
            pltpu.VMEM((96, D), jnp.float32),
            pltpu.VMEM_SHARED((SIZE_P, D), jnp.float32),
            pltpu.SemaphoreType.DMA,
            pltpu.SemaphoreType.DMA,
            pltpu.SemaphoreType.DMA,
        ],
        **_SC_PARAMS,
    )(edge_index, mv16)

    bmax = NBLK // NW + 1
    dsi8, dnv = pl.kernel(
        _norm_body,
        out_type=[
            jax.ShapeDtypeStruct((SIZE_P, 8), jnp.float32),
            jax.ShapeDtypeStruct((NBLK * D, 128), jnp.float32),
        ],
        mesh=mesh,
        scratch_types=[
            pltpu.VMEM((bmax * 128, D), jnp.float32),
            pltpu.VMEM((bmax * 128, D), jnp.float32),
            pltpu.VMEM((bmax * 128, 8), jnp.float32),
            pltpu.VMEM((bmax * D, 128), jnp.float32),
        ],
        **_SC_PARAMS,
    )(part)

    trilv = pl.kernel(
        _gather_body,
        out_type=jax.ShapeDtypeStruct((half * D // 16, 16), jnp.float32),
        mesh=mesh,
        scratch_types=[
            pltpu.VMEM((2, CED), jnp.int32),
            pltpu.VMEM((2, CED), jnp.int32),
            pltpu.VMEM((CED, 8), jnp.float32),
            pltpu.VMEM((CED, 8), jnp.float32),
            pltpu.VMEM((CED, 8), jnp.float32),
            pltpu.VMEM((CED, 8), jnp.float32),
            pltpu.VMEM((CED // 4, 16), jnp.float32),
            pltpu.VMEM((CED // 4, 16), jnp.float32),
            pltpu.VMEM((CED // 4, 16), jnp.float32),
            pltpu.VMEM((CED // 4, 16), jnp.float32),
            pltpu.VMEM((CED // 4, 16), jnp.float32),
            pltpu.VMEM((CED // 4, 16), jnp.float32),
            pltpu.VMEM((2, 64), jnp.int32),
            pltpu.SemaphoreType.DMA,
            pltpu.SemaphoreType.DMA,
            pltpu.SemaphoreType.DMA,
            pltpu.SemaphoreType.DMA,
            pltpu.SemaphoreType.DMA,
            pltpu.SemaphoreType.DMA,
        ],
        **_SC_PARAMS,
    )(edge_index, mv16, dsi8)

    dn = dnv.reshape(NBLK, D, 128).transpose(0, 2, 1).reshape(SIZE_P, D)
    tril = (trilv.reshape(half // 128, D, 128).transpose(0, 2, 1)
            .reshape(half, D))
    return jnp.concatenate([dn[:NUM_NODES], tril], axis=0)


# speedup vs baseline: 23.0657x; 1.0400x over previous
import jax
import jax.numpy as jnp
from jax import lax
from jax.experimental import pallas as pl
from jax.experimental.pallas import tpu as pltpu
from jax.experimental.pallas import tpu_sc as plsc

NUM_NODES = 50000
D = 4
NC, NS = 2, 16
NW = NC * NS

RPT = 3136
SIZE_P = NS * RPT
NBLK = SIZE_P // 128

CE = 1920
NCH_B = 26
EPT_B = 49984
CED = 1664
NCH_D = 15
EPT_D = 24992

_SC_PARAMS = dict(
    compiler_params=pltpu.CompilerParams(
        use_tc_tiling_on_sc=False, needs_layout_passes=False),
)


def _wid():
    return lax.axis_index("s") * NC + lax.axis_index("c")


def _ec16(i):
    q = i * 16 + lax.iota(jnp.int32, 16)
    e = jnp.bitwise_or(
        lax.shift_left(lax.shift_right_logical(q, 9), 7),
        jnp.bitwise_and(q, 127))
    c = jnp.bitwise_and(lax.shift_right_logical(q, 7), 3)
    return e, c


def _scatter_body(e2, mv16, part, idx0, idx1, stg16, vals0, vals1, tidx,
                  tstg, tvals, acc_sh, sem_st, sem_s0, sem_s1):
    cid = lax.axis_index("c")
    sid = lax.axis_index("s")
    wid = sid * NC + cid
    base = wid * EPT_B + 32 * jnp.minimum(wid, NW // 2)

    def zb(i, k):
        e, c = _ec16(i)
        plsc.store_scatter(vals0, [e, c], jnp.zeros((16,), jnp.float32))
        return k
    lax.fori_loop(0, (CE * D) // 16, zb, 0)
    pltpu.sync_copy(vals0, acc_sh.at[pl.ds(sid * RPT, CE)])
    pltpu.sync_copy(vals0.at[pl.ds(0, RPT - CE)],
                    acc_sh.at[pl.ds(sid * RPT + CE, RPT - CE)])
    plsc.subcore_barrier()

    def sq(vals_b, ne):
        def body(i, k):
            e, c = _ec16(i)
            x = stg16[i]
            plsc.store_scatter(vals_b, [e, c], x * x)
            return k
        lax.fori_loop(0, (ne * D) // 16, body, 0, unroll=4)

    def stage(e0, idx_b):
        pltpu.async_copy(e2.at[0, pl.ds(e0, CE)], idx_b, sem_st)
        pltpu.async_copy(mv16.at[pl.ds(e0 // 4, CE // 4)], stg16, sem_st)

    def wait_stage(idx_b):
        pltpu.make_async_copy(e2.at[0, pl.ds(0, CE)], idx_b, sem_st).wait()
        pltpu.make_async_copy(
            mv16.at[pl.ds(0, CE // 4)], stg16, sem_st).wait()

    stage(base, idx0)

    def pair(t, carry):
        k0 = 2 * t
        wait_stage(idx0)
        sq(vals0, CE)

        @pl.when(t > 0)
        def _():
            pltpu.make_async_copy(vals1, acc_sh.at[idx1], sem_s1).wait()
        pltpu.async_copy(vals0, acc_sh.at[idx0], sem_s0, add=True)
        stage(base + (k0 + 1) * CE, idx1)

        wait_stage(idx1)
        sq(vals1, CE)
        pltpu.make_async_copy(vals0, acc_sh.at[idx0], sem_s0).wait()
        pltpu.async_copy(vals1, acc_sh.at[idx1], sem_s1, add=True)

        @pl.when(t < NCH_B // 2 - 1)
        def _():
            stage(base + (k0 + 2) * CE, idx0)
        return carry
    lax.fori_loop(0, NCH_B // 2, pair, 0)
    pltpu.make_async_copy(vals1, acc_sh.at[idx1], sem_s1).wait()

    e0t = base + NCH_B * CE

    def tail(ne):
        pltpu.sync_copy(e2.at[0, pl.ds(e0t, ne)], tidx.at[pl.ds(0, ne)])
        pltpu.sync_copy(mv16.at[pl.ds(e0t // 4, ne // 4)],
                        tstg.at[pl.ds(0, ne // 4)])

        def body(i, k):
            q = i * 16 + lax.iota(jnp.int32, 16)
            e = lax.shift_right_logical(q, 2)
            c = jnp.bitwise_and(q, 3)
            x = tstg[i]
            plsc.store_scatter(tvals, [e, c], x * x)
            return k
        lax.fori_loop(0, (ne * D) // 16, body, 0, unroll=4)
        pltpu.sync_copy(tvals.at[pl.ds(0, ne)],
                        acc_sh.at[tidx.at[pl.ds(0, ne)]], add=True)

    @pl.when(wid < NW // 2)
    def _():
        tail(96)

    @pl.when(wid >= NW // 2)
    def _():
        tail(64)

    plsc.subcore_barrier()
    pltpu.sync_copy(acc_sh.at[pl.ds(sid * RPT, CE)], vals0)
    pltpu.sync_copy(vals0, part.at[cid, pl.ds(sid * RPT, CE)])
    pltpu.sync_copy(acc_sh.at[pl.ds(sid * RPT + CE, RPT - CE)],
                    vals1.at[pl.ds(0, RPT - CE)])
    pltpu.sync_copy(vals1.at[pl.ds(0, RPT - CE)],
                    part.at[cid, pl.ds(sid * RPT + CE, RPT - CE)])


def _rsqrt_newton(x):
    xi = plsc.bitcast(x, jnp.int32)
    y = plsc.bitcast(
        jnp.int32(0x5F3759DF) - lax.shift_right_logical(xi, 1), jnp.float32)
    hx = x * 0.5
    for _ in range(3):
        y = y * (1.5 - hx * y * y)
    return y


def _norm_body(part, dsi8, dnv, p0_v, p1_v, dsi_v, dn_v):
    bpt = NBLK // NW
    brem = NBLK - bpt * NW
    wid = _wid()
    b0 = wid * bpt + jnp.minimum(wid, brem)
    nb = bpt + jnp.where(wid < brem, 1, 0)
    n0 = b0 * 128
    nn = nb * 128

    pltpu.sync_copy(part.at[0, pl.ds(n0, nn)], p0_v.at[pl.ds(0, nn)])
    pltpu.sync_copy(part.at[1, pl.ds(n0, nn)], p1_v.at[pl.ds(0, nn)])

    def body(i, k):
        e, c = _ec16(i)
        q = i * 16 + lax.iota(jnp.int32, 16)
        br = lax.shift_right_logical(q, 7)
        bc = jnp.bitwise_and(q, 127)
        d = (plsc.load_gather(p0_v, [e, c])
             + plsc.load_gather(p1_v, [e, c]))
        dp1 = d + 1.0
        plsc.store_scatter(dsi_v, [e, c], _rsqrt_newton(dp1))
        plsc.store_scatter(dn_v, [br, bc], d / dp1)
        return k
    lax.fori_loop(0, (nn * D) // 16, body, 0)

    pltpu.sync_copy(dsi_v.at[pl.ds(0, nn)], dsi8.at[pl.ds(n0, nn)])
    pltpu.sync_copy(dn_v.at[pl.ds(0, nb * D)], dnv.at[pl.ds(b0 * D, nb * D)])


def _gather_body(e2, mv16, dsi8, trilv, idx0, idx1, bufr0, bufc0, bufr1,
                 bufc1, stgl0, stgr0, stgl1, stgr1, out0, out1, tidx,
                 sem_ix, sem_g0, sem_g1, sem_st0, sem_st1, sem_w):
    wid = _wid()
    half_e = e2.shape[1] // 2
    base = wid * EPT_D + 32 * jnp.minimum(wid, 8)

    idxs = (idx0, idx1)
    bufrs = (bufr0, bufr1)
    bufcs = (bufc0, bufc1)
    stgls = (stgl0, stgl1)
    stgrs = (stgr0, stgr1)
    outs = (out0, out1)
    sem_gs = (sem_g0, sem_g1)
    sem_sts = (sem_st0, sem_st1)

    def stage_idx(e0, b):
        pltpu.async_copy(e2.at[0, pl.ds(e0, CED)], idxs[b].at[0], sem_ix)
        pltpu.async_copy(e2.at[1, pl.ds(e0, CED)], idxs[b].at[1], sem_ix)

    def wait_idx(b):
        pltpu.make_async_copy(
            e2.at[0, pl.ds(0, CED)], idxs[b].at[0], sem_ix).wait()
        pltpu.make_async_copy(
            e2.at[1, pl.ds(0, CED)], idxs[b].at[1], sem_ix).wait()

    def fire_gather(b):
        pltpu.async_copy(dsi8.at[idxs[b].at[0]], bufrs[b], sem_gs[b])
        pltpu.async_copy(dsi8.at[idxs[b].at[1]], bufcs[b], sem_gs[b])

    def drain_gather(b):
        pltpu.make_async_copy(
            dsi8.at[idxs[b].at[0]], bufrs[b], sem_gs[b]).wait()
        pltpu.make_async_copy(
            dsi8.at[idxs[b].at[1]], bufcs[b], sem_gs[b]).wait()

    def stage_data(e0, b):
        pltpu.async_copy(mv16.at[pl.ds(e0 // 4, CED // 4)],
                         stgls[b], sem_sts[b])
        pltpu.async_copy(mv16.at[pl.ds((half_e + e0) // 4, CED // 4)],
                         stgrs[b], sem_sts[b])

    def wait_data(b):
        pltpu.make_async_copy(
            mv16.at[pl.ds(0, CED // 4)], stgls[b], sem_sts[b]).wait()
        pltpu.make_async_copy(
            mv16.at[pl.ds(0, CED // 4)], stgrs[b], sem_sts[b]).wait()

    def mul(b, ne):
        def body(i, k):
            e, c = _ec16(i)
            a = stgls[b][i]
            bb = stgrs[b][i]
            rr = plsc.load_gather(bufrs[b], [e, c])
            cc2 = plsc.load_gather(bufcs[b], [e, c])
            outs[b][i] = -(a * bb) * rr * cc2
            return k
        lax.fori_loop(0, (ne * D) // 16, body, 0, unroll=4)

    def wb_start(e0, b):
        pltpu.async_copy(outs[b], trilv.at[pl.ds(e0 // 4, CED // 4)], sem_w)

    def wb_drain(b):
        pltpu.make_async_copy(
            outs[b], trilv.at[pl.ds(0, CED // 4)], sem_w).wait()

    pltpu.sync_copy(e2.at[0, pl.ds(base, CED)], idx0.at[0])
    pltpu.sync_copy(e2.at[1, pl.ds(base, CED)], idx0.at[1])
    fire_gather(0)
    stage_data(base, 0)
    stage_idx(base + CED, 1)

    def half(t, k, b):
        wait_idx(1 - b)
        fire_gather(1 - b)
        stage_data(base + (k + 1) * CED, 1 - b)
        drain_gather(b)
        wait_data(b)

        @pl.when(k + 2 < NCH_D)
        def _():
            stage_idx(base + (k + 2) * CED, b)

        @pl.when(t > 0)
        def _():
            wb_drain(b)
        mul(b, CED)
        wb_start(base + k * CED, b)

    def pair(t, carry):
        half(t, 2 * t, 0)
        half(t, 2 * t + 1, 1)
        return carry
    lax.fori_loop(0, NCH_D // 2, pair, 0)

    drain_gather(0)
    wait_data(0)
    wb_drain(0)
    mul(0, CED)
    wb_drain(1)
    wb_start(base + (NCH_D - 1) * CED, 0)

    e0t = base + NCH_D * CED

    def tail(ne):
        pltpu.sync_copy(e2.at[0, pl.ds(e0t, ne)], tidx.at[0].at[pl.ds(0, ne)])
        pltpu.sync_copy(e2.at[1, pl.ds(e0t, ne)], tidx.at[1].at[pl.ds(0, ne)])
        pltpu.async_copy(dsi8.at[tidx.at[0].at[pl.ds(0, ne)]],
                         bufr1.at[pl.ds(0, ne)], sem_g1).wait()
        pltpu.async_copy(dsi8.at[tidx.at[1].at[pl.ds(0, ne)]],
                         bufc1.at[pl.ds(0, ne)], sem_g1).wait()
        pltpu.sync_copy(mv16.at[pl.ds(e0t // 4, ne // 4)],
                        stgl1.at[pl.ds(0, ne // 4)])
        pltpu.sync_copy(mv16.at[pl.ds((half_e + e0t) // 4, ne // 4)],
                        stgr1.at[pl.ds(0, ne // 4)])

        def body(i, k):
            q = i * 16 + lax.iota(jnp.int32, 16)
            e = lax.shift_right_logical(q, 2)
            c = jnp.bitwise_and(q, 3)
            a = stgl1[i]
            bb = stgr1[i]
            rr = plsc.load_gather(bufr1, [e, c])
            cc2 = plsc.load_gather(bufc1, [e, c])
            out1[i] = -(a * bb) * rr * cc2
            return k
        lax.fori_loop(0, (ne * D) // 16, body, 0, unroll=4)
        pltpu.sync_copy(out1.at[pl.ds(0, ne // 4)],
                        trilv.at[pl.ds(e0t // 4, ne // 4)])

    @pl.when(wid < 8)
    def _():
        tail(64)

    @pl.when(wid >= 8)
    def _():
        tail(32)

    wb_drain(0)


@jax.jit
def kernel(maps, edge_index):
    half = edge_index.shape[1] // 2
    nblk_e = (2 * half) // 128
    mv16 = (maps.reshape(nblk_e, 128, D).transpose(0, 2, 1)
            .reshape(nblk_e * D * 8, 16))
    mesh = plsc.VectorSubcoreMesh(core_axis_name="c", subcore_axis_name="s")

    part = pl.kernel(
        _scatter_body,
        out_type=jax.ShapeDtypeStruct((NC, SIZE_P, D), jnp.float32),
        mesh=mesh,
        scratch_types=[
            pltpu.VMEM((CE,), jnp.int32),
            pltpu.VMEM((CE,), jnp.int32),
            pltpu.VMEM((CE // 4, 16), jnp.float32),
            pltpu.VMEM((CE, D), jnp.float32),
            pltpu.VMEM((CE, D), jnp.float32),
            pltpu.VMEM((96,), jnp.int32),
            pltpu.VMEM((24, 16), jnp.float32),
            pltpu.VMEM((96, D), jnp.float32),
            pltpu.VMEM_SHARED((SIZE_P, D), jnp.float32),
            pltpu.SemaphoreType.DMA,
            pltpu.SemaphoreType.DMA,
            pltpu.SemaphoreType.DMA,
        ],
        **_SC_PARAMS,
    )(edge_index, mv16)

    bmax = NBLK // NW + 1
    dsi8, dnv = pl.kernel(
        _norm_body,
        out_type=[
            jax.ShapeDtypeStruct((SIZE_P, 8), jnp.float32),
            jax.ShapeDtypeStruct((NBLK * D, 128), jnp.float32),
        ],
        mesh=mesh,
        scratch_types=[
            pltpu.VMEM((bmax * 128, D), jnp.float32),
            pltpu.VMEM((bmax * 128, D), jnp.float32),
            pltpu.VMEM((bmax * 128, 8), jnp.float32),
            pltpu.VMEM((bmax * D, 128), jnp.float32),
        ],
        **_SC_PARAMS,
    )(part)

    trilv = pl.kernel(
        _gather_body,
        out_type=jax.ShapeDtypeStruct((half * D // 16, 16), jnp.float32),
        mesh=mesh,
        scratch_types=[
            pltpu.VMEM((2, CED), jnp.int32),
            pltpu.VMEM((2, CED), jnp.int32),
            pltpu.VMEM((CED, 8), jnp.float32),
            pltpu.VMEM((CED, 8), jnp.float32),
            pltpu.VMEM((CED, 8), jnp.float32),
            pltpu.VMEM((CED, 8), jnp.float32),
            pltpu.VMEM((CED // 4, 16), jnp.float32),
            pltpu.VMEM((CED // 4, 16), jnp.float32),
            pltpu.VMEM((CED // 4, 16), jnp.float32),
            pltpu.VMEM((CED // 4, 16), jnp.float32),
            pltpu.VMEM((CED // 4, 16), jnp.float32),
            pltpu.VMEM((CED // 4, 16), jnp.float32),
            pltpu.VMEM((2, 64), jnp.int32),
            pltpu.SemaphoreType.DMA,
            pltpu.SemaphoreType.DMA,
            pltpu.SemaphoreType.DMA,
            pltpu.SemaphoreType.DMA,
            pltpu.SemaphoreType.DMA,
            pltpu.SemaphoreType.DMA,
        ],
        **_SC_PARAMS,
    )(edge_index, mv16, dsi8)

    dn = dnv.reshape(NBLK, D, 128).transpose(0, 2, 1).reshape(SIZE_P, D)
    tril = (trilv.reshape(half // 128, D, 128).transpose(0, 2, 1)
            .reshape(half, D))
    return jnp.concatenate([dn[:NUM_NODES], tril], axis=0)
